# baseline (speedup 1.0000x reference)
import functools

import jax
import jax.numpy as jnp
from jax import lax
from jax.experimental import pallas as pl
from jax.experimental.pallas import tpu as pltpu
from jax.experimental.pallas import tpu_sc as plsc

B = 32
NPG = 500
N = B * NPG
EPG = 2000
E = B * EPG
NC = 100
HALF = NPG * NPG // 2
ACC = 125056
QTOT = 35
GPB = 4


def _sc_build_adj(src, dst, w):
    info = plsc.get_sparse_core_info()
    nc = info.num_cores
    mesh = plsc.VectorSubcoreMesh(core_axis_name="c", subcore_axis_name="s")

    @functools.partial(
        pl.kernel,
        mesh=mesh,
        compiler_params=pltpu.CompilerParams(needs_layout_passes=False),
        out_type=[
            jax.ShapeDtypeStruct((B * NPG * NPG,), jnp.float32),
            jax.ShapeDtypeStruct((B * NPG * NPG,), jnp.float32),
        ],
        scratch_types=[
            pltpu.VMEM((ACC,), jnp.float32),
            pltpu.VMEM((EPG,), jnp.int32),
            pltpu.VMEM((EPG,), jnp.int32),
        ],
    )
    def build(src_hbm, dst_hbm, wbits_hbm, adj_hbm, adj1_hbm, acc, eidx, sbuf):
        g = lax.axis_index("s") * nc + lax.axis_index("c")
        base_e = g * EPG
        pltpu.sync_copy(src_hbm.at[pl.ds(base_e, EPG)], eidx)
        pltpu.sync_copy(dst_hbm.at[pl.ds(base_e, EPG)], sbuf)
        node0 = g * NPG

        def idx_body(c, carry):
            ss = eidx[pl.ds(c * 16, 16)]
            dd = sbuf[pl.ds(c * 16, 16)]
            eidx[pl.ds(c * 16, 16)] = (ss - node0) * NPG + (dd - node0)
            return carry

        lax.fori_loop(0, EPG // 16, idx_body, 0)
        pltpu.sync_copy(wbits_hbm.at[pl.ds(base_e, EPG)], sbuf)

        lanes = lax.iota(jnp.int32, 16)
        zeros16 = jnp.zeros((16,), jnp.float32)
        ones16 = jnp.ones((16,), jnp.float32)

        for out_hbm, use_w in ((adj_hbm, True), (adj1_hbm, False)):
            for part in range(2):
                def zero_body(i, carry):
                    for u in range(8):
                        acc[pl.ds(i * 128 + u * 16, 16)] = zeros16
                    return carry

                lax.fori_loop(0, ACC // 128, zero_body, 0)
                lo = part * HALF

                def scat_body(c, carry):
                    fl = eidx[pl.ds(c * 16, 16)]
                    loc = fl - lo
                    inb = (loc >= 0) & (loc < HALF)
                    loc = jnp.where(inb, loc, 0)
                    if use_w:
                        vals = plsc.bitcast(sbuf[pl.ds(c * 16, 16)], jnp.float32)
                    else:
                        vals = ones16
                    for j in range(16):
                        plsc.addupdate_scatter(
                            acc, [loc], vals, mask=inb & (lanes == j)
                        )
                    return carry

                lax.fori_loop(0, EPG // 16, scat_body, 0)
                pltpu.sync_copy(
                    acc.at[pl.ds(0, HALF)],
                    out_hbm.at[pl.ds(g * NPG * NPG + lo, HALF)],
                )

    adj, adj1 = build(src, dst, lax.bitcast_convert_type(w, jnp.int32))
    return adj.reshape(B, NPG, NPG), adj1.reshape(B, NPG, NPG)


_CT_LHS = (((0,), (0,)), ((), ()))
_CT_STD = (((1,), (0,)), ((), ()))
_CT_RHS = (((1,), (1,)), ((), ()))
_HI = jax.lax.Precision.HIGHEST


def _dot(a, b, ct):
    return lax.dot_general(a, b, ct, precision=_HI,
                           preferred_element_type=jnp.float32)


def _full(a):
    return pl.BlockSpec(a.shape, lambda g: (0,) * len(a.shape))


def _pg(shape):
    return pl.BlockSpec((GPB,) + shape, lambda g: (g, 0, 0))


def _vspec(f):
    return pl.BlockSpec((f, 1), lambda g: (0, 0))


def _bn_coefs_t(sum_v, sq_v, gamma, beta, count):
    s = jnp.sum(sum_v, axis=0)
    ss = jnp.sum(sq_v, axis=0)
    m = s / count
    v = ss / count - m * m
    inv = lax.rsqrt(v + 1e-5)
    scale = gamma * inv
    shift = beta - m * scale
    return scale, shift


def _gcn_core_t(ht, A, W, b):
    hwt = _dot(W, ht, _CT_LHS)
    deg = jnp.sum(A, axis=0, keepdims=True) + 1.0
    dinv = lax.rsqrt(jnp.maximum(deg, 1e-12))
    aggt = _dot(hwt * dinv, A, _CT_STD)
    return aggt * dinv + (dinv * dinv) * hwt + b


def _stats_out_t(out):
    s = jnp.sum(out, axis=1, keepdims=True)
    q = jnp.sum(out * out, axis=1, keepdims=True)
    return s, q


def _gcn_pair(hx, hs, adj, adj1, Wx, bx, Ws, bs, bn_x=None, bn_s=None):
    finx, foutx = Wx.shape
    fins, fouts = Ws.shape
    with_bn = bn_x is not None

    def body(*refs):
        if with_bn:
            (hx_ref, hs_ref, a_ref, a1_ref, wx_ref, bx_ref, ws_ref, bs_ref,
             sx_ref, qx_ref, gx_ref, bex_ref,
             ss_ref, qs_ref, gs_ref, bes_ref,
             ox_ref, osx_ref, oqx_ref, os_ref, oss_ref, oqs_ref) = refs
            scx, shx = _bn_coefs_t(sx_ref[...], qx_ref[...], gx_ref[...],
                                   bex_ref[...], N)
            scs, shs = _bn_coefs_t(ss_ref[...], qs_ref[...], gs_ref[...],
                                   bes_ref[...], N)
        else:
            (hx_ref, hs_ref, a_ref, a1_ref, wx_ref, bx_ref, ws_ref, bs_ref,
             ox_ref, osx_ref, oqx_ref, os_ref, oss_ref, oqs_ref) = refs
        for i in range(GPB):
            hxi = hx_ref[i]
            hsi = hs_ref[i]
            if with_bn:
                hxi = hxi * scx + shx
                hsi = hsi * scs + shs
            ox = _gcn_core_t(hxi, a_ref[i], wx_ref[...], bx_ref[...])
            ox_ref[i] = ox
            osx_ref[i], oqx_ref[i] = _stats_out_t(ox)
            os_ = _gcn_core_t(hsi, a1_ref[i], ws_ref[...], bs_ref[...])
            os_ref[i] = os_
            oss_ref[i], oqs_ref[i] = _stats_out_t(os_)

    in_specs = [_pg((finx, NPG)), _pg((fins, NPG)),
                _pg((NPG, NPG)), _pg((NPG, NPG)),
                _full(Wx), _vspec(foutx), _full(Ws), _vspec(fouts)]
    args = [hx, hs, adj, adj1, Wx, bx.reshape(foutx, 1),
            Ws, bs.reshape(fouts, 1)]
    if with_bn:
        sx, qx, gx, bex = bn_x
        ss, qs, gs, bes = bn_s
        in_specs += [_full(sx), _full(qx), _vspec(finx), _vspec(finx),
                     _full(ss), _full(qs), _vspec(fins), _vspec(fins)]
        args += [sx, qx, gx.reshape(finx, 1), bex.reshape(finx, 1),
                 ss, qs, gs.reshape(fins, 1), bes.reshape(fins, 1)]

    out_shapes = (jax.ShapeDtypeStruct((B, foutx, NPG), jnp.float32),
                  jax.ShapeDtypeStruct((B, foutx, 1), jnp.float32),
                  jax.ShapeDtypeStruct((B, foutx, 1), jnp.float32),
                  jax.ShapeDtypeStruct((B, fouts, NPG), jnp.float32),
                  jax.ShapeDtypeStruct((B, fouts, 1), jnp.float32),
                  jax.ShapeDtypeStruct((B, fouts, 1), jnp.float32))
    out_specs = (_pg((foutx, NPG)), _pg((foutx, 1)), _pg((foutx, 1)),
                 _pg((fouts, NPG)), _pg((fouts, 1)), _pg((fouts, 1)))
    return pl.pallas_call(
        body, grid=(B // GPB,), in_specs=in_specs, out_specs=out_specs,
        out_shape=out_shapes)(*args)


_BP_GROUPS = []
_off = 0
for _q in range(2, 9):
    _BP_GROUPS.append((_off, _q))
    _off += _q


def _bp_pool(adj, bel0t, xs, xstats, sgs, sgstats, p):
    x11, x12, x13 = xs
    sg11, sg12, sg13 = sgs

    def body(a_ref, b0_ref,
             x11_ref, x12_ref, x13_ref,
             xs1_ref, xq1_ref, xs2_ref, xq2_ref, xs3_ref, xq3_ref,
             g1_ref, be1_ref, g2_ref, be2_ref, g3_ref, be3_ref,
             sg1_ref, sg2_ref, sg3_ref,
             ss1_ref, sq1_ref, ss2_ref, sq2_ref, ss3_ref, sq3_ref,
             h1_ref, hb1_ref, h2_ref, hb2_ref, h3_ref, hb3_ref,
             wpf_ref, bpf_ref,
             p1x_ref, p1a_ref, x1o_ref):
        sc1, sh1 = _bn_coefs_t(xs1_ref[...], xq1_ref[...], g1_ref[...], be1_ref[...], N)
        sc2, sh2 = _bn_coefs_t(xs2_ref[...], xq2_ref[...], g2_ref[...], be2_ref[...], N)
        sc3, sh3 = _bn_coefs_t(xs3_ref[...], xq3_ref[...], g3_ref[...], be3_ref[...], N)
        t1, u1 = _bn_coefs_t(ss1_ref[...], sq1_ref[...], h1_ref[...], hb1_ref[...], N)
        t2, u2 = _bn_coefs_t(ss2_ref[...], sq2_ref[...], h2_ref[...], hb2_ref[...], N)
        t3, u3 = _bn_coefs_t(ss3_ref[...], sq3_ref[...], h3_ref[...], hb3_ref[...], N)
        for i in range(GPB):
            A = a_ref[i]
            bel = b0_ref[i]
            for _ in range(10):
                msg = _dot(bel, A, _CT_STD)
                z = jnp.log(bel + 1e-9) + msg
                parts = []
                for off, q in _BP_GROUPS:
                    zq = z[off:off + q, :]
                    zq = zq - jnp.max(zq, axis=0, keepdims=True)
                    e = jnp.exp(zq)
                    parts.append(e / jnp.sum(e, axis=0, keepdims=True))
                bel = jnp.concatenate(parts, axis=0)
            x11n = x11_ref[i] * sc1 + sh1
            x12n = x12_ref[i] * sc2 + sh2
            x13n = x13_ref[i] * sc3 + sh3
            sg1n = sg1_ref[i] * t1 + u1
            sg2n = sg2_ref[i] * t2 + u2
            sg3n = sg3_ref[i] * t3 + u3
            feat = jnp.concatenate([sg1n, sg2n, sg3n, bel], axis=0)
            s1 = _dot(wpf_ref[...], feat, _CT_LHS) + bpf_ref[...]
            s1 = s1 - jnp.max(s1, axis=0, keepdims=True)
            es = jnp.exp(s1)
            st = es / jnp.sum(es, axis=0, keepdims=True)
            p1x_ref[i] = _dot(st, x13n, _CT_RHS)
            H = _dot(st, A, _CT_STD)
            p1a_ref[i] = _dot(H, st, _CT_RHS)
            x1cat = jnp.concatenate([x11n, x12n, x13n], axis=0)
            x1o_ref[i] = jnp.max(x1cat, axis=1).reshape(1, 90)

    in_specs = [_pg((NPG, NPG)), _pg((QTOT, NPG))]
    in_specs += [_pg((30, NPG))] * 3
    in_specs += [_full(xstats[0][0])] * 6
    in_specs += [_vspec(30)] * 6
    in_specs += [_pg((30, NPG)), _pg((30, NPG)), _pg((100, NPG))]
    in_specs += [_full(sgstats[0][0]), _full(sgstats[0][1]),
                 _full(sgstats[1][0]), _full(sgstats[1][1]),
                 _full(sgstats[2][0]), _full(sgstats[2][1])]
    in_specs += [_vspec(30), _vspec(30), _vspec(30), _vspec(30),
                 _vspec(100), _vspec(100)]
    in_specs += [_full(p["Wpf"]), _vspec(100)]

    args = [adj, bel0t, x11, x12, x13]
    args += [xstats[0][0], xstats[0][1], xstats[1][0], xstats[1][1],
             xstats[2][0], xstats[2][1]]
    args += [p["gn11"].reshape(30, 1), p["ben11"].reshape(30, 1),
             p["gn12"].reshape(30, 1), p["ben12"].reshape(30, 1),
             p["gn13"].reshape(30, 1), p["ben13"].reshape(30, 1)]
    args += [sg11, sg12, sg13]
    args += [sgstats[0][0], sgstats[0][1], sgstats[1][0], sgstats[1][1],
             sgstats[2][0], sgstats[2][1]]
    args += [p["gnp11"].reshape(30, 1), p["benp11"].reshape(30, 1),
             p["gnp12"].reshape(30, 1), p["benp12"].reshape(30, 1),
             p["gnp13"].reshape(100, 1), p["benp13"].reshape(100, 1)]
    args += [p["Wpf"], p["bpf"].reshape(100, 1)]

    out_shapes = (jax.ShapeDtypeStruct((B, NC, 30), jnp.float32),
                  jax.ShapeDtypeStruct((B, NC, NC), jnp.float32),
                  jax.ShapeDtypeStruct((B, 1, 90), jnp.float32))
    out_specs = (_pg((NC, 30)), _pg((NC, NC)), _pg((1, 90)))
    return pl.pallas_call(
        body, grid=(B // GPB,), in_specs=in_specs, out_specs=out_specs,
        out_shape=out_shapes)(*args)


def _dense_head(p1x, p1adj, x1out, p):
    def bn_full(h, gamma, beta):
        cnt = B * NC
        m = jnp.sum(h, axis=(0, 1)).reshape(1, 1, -1) / cnt
        v = jnp.sum(h * h, axis=(0, 1)).reshape(1, 1, -1) / cnt - m * m
        inv = lax.rsqrt(v + 1e-5)
        return (h - m) * inv * gamma + beta

    def body(p1x_ref, p1a_ref, x1o_ref,
             w21_ref, b21_ref, w22_ref, b22_ref, w23_ref, b23_ref,
             g21_ref, be21_ref, g22_ref, be22_ref, g23_ref, be23_ref,
             w1_ref, b1_ref, w2_ref, b2_ref, out_ref):
        eye = (lax.broadcasted_iota(jnp.int32, (NC, NC), 0)
               == lax.broadcasted_iota(jnp.int32, (NC, NC), 1))
        a = p1a_ref[...] + eye.astype(jnp.float32)[None]
        deg = jnp.sum(a, axis=2, keepdims=True)
        dinv = lax.rsqrt(jnp.maximum(deg, 1e-12))
        an_rows = [dinv[g] * a[g] * dinv[g].reshape(1, NC) for g in range(B)]

        def dense_layer(h, w, bias):
            hw = _dot(h.reshape(B * NC, -1), w, _CT_STD) + bias
            hw = hw.reshape(B, NC, -1)
            rows = [_dot(an_rows[g], hw[g], _CT_STD)[None] for g in range(B)]
            return jnp.concatenate(rows, axis=0)

        x21 = dense_layer(p1x_ref[...], w21_ref[...], b21_ref[...])
        x21n = bn_full(x21, g21_ref[...], be21_ref[...])
        x22 = dense_layer(x21n, w22_ref[...], b22_ref[...])
        x22n = bn_full(x22, g22_ref[...], be22_ref[...])
        x23 = dense_layer(x22n, w23_ref[...], b23_ref[...])
        x23n = bn_full(x23, g23_ref[...], be23_ref[...])
        x2 = jnp.concatenate([x21n, x22n, x23n], axis=2)
        x2out = jnp.max(x2, axis=1)
        conv = jnp.concatenate([x1o_ref[...].reshape(B, 90), x2out], axis=1)
        h = _dot(conv, w1_ref[...], _CT_STD) + b1_ref[...]
        h = jnp.maximum(h, 0.0)
        out_ref[...] = _dot(h, w2_ref[...], _CT_STD) + b2_ref[...]

    args = [p1x, p1adj, x1out,
            p["W21"], p["b21"].reshape(1, 30),
            p["W22"], p["b22"].reshape(1, 30),
            p["W23"], p["b23"].reshape(1, 30),
            p["gn21"].reshape(1, 1, 30), p["ben21"].reshape(1, 1, 30),
            p["gn22"].reshape(1, 1, 30), p["ben22"].reshape(1, 1, 30),
            p["gn23"].reshape(1, 1, 30), p["ben23"].reshape(1, 1, 30),
            p["Wf1"], p["bf1"].reshape(1, 50),
            p["Wf2"], p["bf2"].reshape(1, 6)]
    return pl.pallas_call(
        body,
        out_shape=jax.ShapeDtypeStruct((B, 6), jnp.float32),
    )(*args)


def kernel(x, edge_index, edge_attr, params):
    p = params
    src = edge_index[0]
    dst = edge_index[1]
    adj, adj1 = _sc_build_adj(src, dst, edge_attr)

    x3t = x.reshape(B, NPG, 3).swapaxes(1, 2)

    x11, xs1, xq1, sg11, ss1, sq1 = _gcn_pair(
        x3t, x3t, adj, adj1, p["W11"], p["b11"], p["Wp11"], p["bp11"])
    x12, xs2, xq2, sg12, ss2, sq2 = _gcn_pair(
        x11, sg11, adj, adj1, p["W12"], p["b12"], p["Wp12"], p["bp12"],
        bn_x=(xs1, xq1, p["gn11"], p["ben11"]),
        bn_s=(ss1, sq1, p["gnp11"], p["benp11"]))
    x13, xs3, xq3, sg13, ss3, sq3 = _gcn_pair(
        x12, sg12, adj, adj1, p["W13"], p["b13"], p["Wp13"], p["bp13"],
        bn_x=(xs2, xq2, p["gn12"], p["ben12"]),
        bn_s=(ss2, sq2, p["gnp12"], p["benp12"]))

    parts = []
    for q in range(2, 9):
        ph = jnp.sin(jnp.arange(N * q, dtype=jnp.float32) * 0.37).reshape(N, q)
        parts.append(jax.nn.softmax(ph, axis=-1))
    bel0t = jnp.concatenate(parts, axis=-1).reshape(B, NPG, QTOT).swapaxes(1, 2)

    p1x, p1adj, x1out = _bp_pool(
        adj, bel0t, (x11, x12, x13),
        ((xs1, xq1), (xs2, xq2), (xs3, xq3)),
        (sg11, sg12, sg13),
        ((ss1, sq1), (ss2, sq2), (ss3, sq3)), p)

    out = _dense_head(p1x, p1adj, x1out, p)
    reg = jnp.zeros((1,), jnp.float32)
    return (out, reg)


# speedup vs baseline: 1.5082x; 1.5082x over previous
import functools

import jax
import jax.numpy as jnp
from jax import lax
from jax.experimental import pallas as pl
from jax.experimental.pallas import tpu as pltpu
from jax.experimental.pallas import tpu_sc as plsc

B = 32
NPG = 500
N = B * NPG
EPG = 2000
E = B * EPG
NC = 100
HALF = NPG * NPG // 2
ACC = 125056
QTOT = 35
GPB = 4


def _sc_build_adj(src, dst, w):
    info = plsc.get_sparse_core_info()
    nc = info.num_cores
    mesh = plsc.VectorSubcoreMesh(core_axis_name="c", subcore_axis_name="s")

    @functools.partial(
        pl.kernel,
        mesh=mesh,
        compiler_params=pltpu.CompilerParams(needs_layout_passes=False),
        out_type=[
            jax.ShapeDtypeStruct((B * NPG * NPG,), jnp.float32),
            jax.ShapeDtypeStruct((B * NPG * NPG,), jnp.float32),
        ],
        scratch_types=[
            pltpu.VMEM((ACC,), jnp.float32),
            pltpu.VMEM((EPG,), jnp.int32),
            pltpu.VMEM((EPG,), jnp.int32),
        ],
    )
    def build(src_hbm, dst_hbm, wbits_hbm, adj_hbm, adj1_hbm, acc, eidx, sbuf):
        g = lax.axis_index("s") * nc + lax.axis_index("c")
        base_e = g * EPG
        pltpu.sync_copy(src_hbm.at[pl.ds(base_e, EPG)], eidx)
        pltpu.sync_copy(dst_hbm.at[pl.ds(base_e, EPG)], sbuf)
        node0 = g * NPG

        def idx_body(c, carry):
            ss = eidx[pl.ds(c * 16, 16)]
            dd = sbuf[pl.ds(c * 16, 16)]
            eidx[pl.ds(c * 16, 16)] = (ss - node0) * NPG + (dd - node0)
            return carry

        lax.fori_loop(0, EPG // 16, idx_body, 0)
        pltpu.sync_copy(wbits_hbm.at[pl.ds(base_e, EPG)], sbuf)

        lanes = lax.iota(jnp.int32, 16)
        zeros16 = jnp.zeros((16,), jnp.float32)
        ones16 = jnp.ones((16,), jnp.float32)

        for out_hbm, use_w in ((adj_hbm, True), (adj1_hbm, False)):
            for part in range(2):
                def zero_body(i, carry):
                    for u in range(8):
                        acc[pl.ds(i * 128 + u * 16, 16)] = zeros16
                    return carry

                lax.fori_loop(0, ACC // 128, zero_body, 0)
                lo = part * HALF

                def scat_body(c, carry):
                    fl = eidx[pl.ds(c * 16, 16)]
                    loc = fl - lo
                    inb = (loc >= 0) & (loc < HALF)
                    loc = jnp.where(inb, loc, 0)
                    if use_w:
                        vals = plsc.bitcast(sbuf[pl.ds(c * 16, 16)], jnp.float32)
                    else:
                        vals = ones16
                    for j in range(16):
                        plsc.addupdate_scatter(
                            acc, [loc], vals, mask=inb & (lanes == j)
                        )
                    return carry

                lax.fori_loop(0, EPG // 16, scat_body, 0)
                pltpu.sync_copy(
                    acc.at[pl.ds(0, HALF)],
                    out_hbm.at[pl.ds(g * NPG * NPG + lo, HALF)],
                )

    adj, adj1 = build(src, dst, lax.bitcast_convert_type(w, jnp.int32))
    return adj.reshape(B, NPG, NPG), adj1.reshape(B, NPG, NPG)


_CT_LHS = (((0,), (0,)), ((), ()))
_CT_STD = (((1,), (0,)), ((), ()))
_CT_RHS = (((1,), (1,)), ((), ()))
_HI = jax.lax.Precision.DEFAULT


def _dot(a, b, ct):
    return lax.dot_general(a, b, ct, precision=_HI,
                           preferred_element_type=jnp.float32)


def _full(a):
    return pl.BlockSpec(a.shape, lambda g: (0,) * len(a.shape))


def _pg(shape):
    return pl.BlockSpec((GPB,) + shape, lambda g: (g, 0, 0))


def _vspec(f):
    return pl.BlockSpec((f, 1), lambda g: (0, 0))


def _bn_coefs_t(sum_v, sq_v, gamma, beta, count):
    s = jnp.sum(sum_v, axis=0)
    ss = jnp.sum(sq_v, axis=0)
    m = s / count
    v = ss / count - m * m
    inv = lax.rsqrt(v + 1e-5)
    scale = gamma * inv
    shift = beta - m * scale
    return scale, shift


def _gcn_core_t(ht, A, W, b):
    hwt = _dot(W, ht, _CT_LHS)
    deg = jnp.sum(A, axis=0, keepdims=True) + 1.0
    dinv = lax.rsqrt(jnp.maximum(deg, 1e-12))
    aggt = _dot(hwt * dinv, A, _CT_STD)
    return aggt * dinv + (dinv * dinv) * hwt + b


def _stats_out_t(out):
    s = jnp.sum(out, axis=1, keepdims=True)
    q = jnp.sum(out * out, axis=1, keepdims=True)
    return s, q


def _gcn_pair(hx, hs, adj, adj1, Wx, bx, Ws, bs, bn_x=None, bn_s=None):
    finx, foutx = Wx.shape
    fins, fouts = Ws.shape
    with_bn = bn_x is not None

    def body(*refs):
        if with_bn:
            (hx_ref, hs_ref, a_ref, a1_ref, wx_ref, bx_ref, ws_ref, bs_ref,
             sx_ref, qx_ref, gx_ref, bex_ref,
             ss_ref, qs_ref, gs_ref, bes_ref,
             ox_ref, osx_ref, oqx_ref, os_ref, oss_ref, oqs_ref) = refs
            scx, shx = _bn_coefs_t(sx_ref[...], qx_ref[...], gx_ref[...],
                                   bex_ref[...], N)
            scs, shs = _bn_coefs_t(ss_ref[...], qs_ref[...], gs_ref[...],
                                   bes_ref[...], N)
        else:
            (hx_ref, hs_ref, a_ref, a1_ref, wx_ref, bx_ref, ws_ref, bs_ref,
             ox_ref, osx_ref, oqx_ref, os_ref, oss_ref, oqs_ref) = refs
        for i in range(GPB):
            hxi = hx_ref[i]
            hsi = hs_ref[i]
            if with_bn:
                hxi = hxi * scx + shx
                hsi = hsi * scs + shs
            ox = _gcn_core_t(hxi, a_ref[i], wx_ref[...], bx_ref[...])
            ox_ref[i] = ox
            osx_ref[i], oqx_ref[i] = _stats_out_t(ox)
            os_ = _gcn_core_t(hsi, a1_ref[i], ws_ref[...], bs_ref[...])
            os_ref[i] = os_
            oss_ref[i], oqs_ref[i] = _stats_out_t(os_)

    in_specs = [_pg((finx, NPG)), _pg((fins, NPG)),
                _pg((NPG, NPG)), _pg((NPG, NPG)),
                _full(Wx), _vspec(foutx), _full(Ws), _vspec(fouts)]
    args = [hx, hs, adj, adj1, Wx, bx.reshape(foutx, 1),
            Ws, bs.reshape(fouts, 1)]
    if with_bn:
        sx, qx, gx, bex = bn_x
        ss, qs, gs, bes = bn_s
        in_specs += [_full(sx), _full(qx), _vspec(finx), _vspec(finx),
                     _full(ss), _full(qs), _vspec(fins), _vspec(fins)]
        args += [sx, qx, gx.reshape(finx, 1), bex.reshape(finx, 1),
                 ss, qs, gs.reshape(fins, 1), bes.reshape(fins, 1)]

    out_shapes = (jax.ShapeDtypeStruct((B, foutx, NPG), jnp.float32),
                  jax.ShapeDtypeStruct((B, foutx, 1), jnp.float32),
                  jax.ShapeDtypeStruct((B, foutx, 1), jnp.float32),
                  jax.ShapeDtypeStruct((B, fouts, NPG), jnp.float32),
                  jax.ShapeDtypeStruct((B, fouts, 1), jnp.float32),
                  jax.ShapeDtypeStruct((B, fouts, 1), jnp.float32))
    out_specs = (_pg((foutx, NPG)), _pg((foutx, 1)), _pg((foutx, 1)),
                 _pg((fouts, NPG)), _pg((fouts, 1)), _pg((fouts, 1)))
    return pl.pallas_call(
        body, grid=(B // GPB,), in_specs=in_specs, out_specs=out_specs,
        out_shape=out_shapes)(*args)


_BP_GROUPS = []
_off = 0
for _q in range(2, 9):
    _BP_GROUPS.append((_off, _q))
    _off += _q


def _bp_pool(adj, bel0t, xs, xstats, sgs, sgstats, p):
    x11, x12, x13 = xs
    sg11, sg12, sg13 = sgs

    def body(a_ref, b0_ref,
             x11_ref, x12_ref, x13_ref,
             xs1_ref, xq1_ref, xs2_ref, xq2_ref, xs3_ref, xq3_ref,
             g1_ref, be1_ref, g2_ref, be2_ref, g3_ref, be3_ref,
             sg1_ref, sg2_ref, sg3_ref,
             ss1_ref, sq1_ref, ss2_ref, sq2_ref, ss3_ref, sq3_ref,
             h1_ref, hb1_ref, h2_ref, hb2_ref, h3_ref, hb3_ref,
             wpf_ref, bpf_ref,
             p1x_ref, p1a_ref, x1o_ref):
        sc1, sh1 = _bn_coefs_t(xs1_ref[...], xq1_ref[...], g1_ref[...], be1_ref[...], N)
        sc2, sh2 = _bn_coefs_t(xs2_ref[...], xq2_ref[...], g2_ref[...], be2_ref[...], N)
        sc3, sh3 = _bn_coefs_t(xs3_ref[...], xq3_ref[...], g3_ref[...], be3_ref[...], N)
        t1, u1 = _bn_coefs_t(ss1_ref[...], sq1_ref[...], h1_ref[...], hb1_ref[...], N)
        t2, u2 = _bn_coefs_t(ss2_ref[...], sq2_ref[...], h2_ref[...], hb2_ref[...], N)
        t3, u3 = _bn_coefs_t(ss3_ref[...], sq3_ref[...], h3_ref[...], hb3_ref[...], N)
        for i in range(GPB):
            A = a_ref[i]
            bel = b0_ref[i]
            for _ in range(10):
                msg = _dot(bel, A, _CT_STD)
                z = jnp.log(bel + 1e-9) + msg
                parts = []
                for off, q in _BP_GROUPS:
                    zq = z[off:off + q, :]
                    zq = zq - jnp.max(zq, axis=0, keepdims=True)
                    e = jnp.exp(zq)
                    parts.append(e / jnp.sum(e, axis=0, keepdims=True))
                bel = jnp.concatenate(parts, axis=0)
            x11n = x11_ref[i] * sc1 + sh1
            x12n = x12_ref[i] * sc2 + sh2
            x13n = x13_ref[i] * sc3 + sh3
            sg1n = sg1_ref[i] * t1 + u1
            sg2n = sg2_ref[i] * t2 + u2
            sg3n = sg3_ref[i] * t3 + u3
            feat = jnp.concatenate([sg1n, sg2n, sg3n, bel], axis=0)
            s1 = _dot(wpf_ref[...], feat, _CT_LHS) + bpf_ref[...]
            s1 = s1 - jnp.max(s1, axis=0, keepdims=True)
            es = jnp.exp(s1)
            st = es / jnp.sum(es, axis=0, keepdims=True)
            p1x_ref[i] = _dot(st, x13n, _CT_RHS)
            H = _dot(st, A, _CT_STD)
            p1a_ref[i] = _dot(H, st, _CT_RHS)
            x1cat = jnp.concatenate([x11n, x12n, x13n], axis=0)
            x1o_ref[i] = jnp.max(x1cat, axis=1).reshape(1, 90)

    in_specs = [_pg((NPG, NPG)), _pg((QTOT, NPG))]
    in_specs += [_pg((30, NPG))] * 3
    in_specs += [_full(xstats[0][0])] * 6
    in_specs += [_vspec(30)] * 6
    in_specs += [_pg((30, NPG)), _pg((30, NPG)), _pg((100, NPG))]
    in_specs += [_full(sgstats[0][0]), _full(sgstats[0][1]),
                 _full(sgstats[1][0]), _full(sgstats[1][1]),
                 _full(sgstats[2][0]), _full(sgstats[2][1])]
    in_specs += [_vspec(30), _vspec(30), _vspec(30), _vspec(30),
                 _vspec(100), _vspec(100)]
    in_specs += [_full(p["Wpf"]), _vspec(100)]

    args = [adj, bel0t, x11, x12, x13]
    args += [xstats[0][0], xstats[0][1], xstats[1][0], xstats[1][1],
             xstats[2][0], xstats[2][1]]
    args += [p["gn11"].reshape(30, 1), p["ben11"].reshape(30, 1),
             p["gn12"].reshape(30, 1), p["ben12"].reshape(30, 1),
             p["gn13"].reshape(30, 1), p["ben13"].reshape(30, 1)]
    args += [sg11, sg12, sg13]
    args += [sgstats[0][0], sgstats[0][1], sgstats[1][0], sgstats[1][1],
             sgstats[2][0], sgstats[2][1]]
    args += [p["gnp11"].reshape(30, 1), p["benp11"].reshape(30, 1),
             p["gnp12"].reshape(30, 1), p["benp12"].reshape(30, 1),
             p["gnp13"].reshape(100, 1), p["benp13"].reshape(100, 1)]
    args += [p["Wpf"], p["bpf"].reshape(100, 1)]

    out_shapes = (jax.ShapeDtypeStruct((B, NC, 30), jnp.float32),
                  jax.ShapeDtypeStruct((B, NC, NC), jnp.float32),
                  jax.ShapeDtypeStruct((B, 1, 90), jnp.float32))
    out_specs = (_pg((NC, 30)), _pg((NC, NC)), _pg((1, 90)))
    return pl.pallas_call(
        body, grid=(B // GPB,), in_specs=in_specs, out_specs=out_specs,
        out_shape=out_shapes)(*args)


def _dense_head(p1x, p1adj, x1out, p):
    def bn_full(h, gamma, beta):
        cnt = B * NC
        m = jnp.sum(h, axis=(0, 1)).reshape(1, 1, -1) / cnt
        v = jnp.sum(h * h, axis=(0, 1)).reshape(1, 1, -1) / cnt - m * m
        inv = lax.rsqrt(v + 1e-5)
        return (h - m) * inv * gamma + beta

    def body(p1x_ref, p1a_ref, x1o_ref,
             w21_ref, b21_ref, w22_ref, b22_ref, w23_ref, b23_ref,
             g21_ref, be21_ref, g22_ref, be22_ref, g23_ref, be23_ref,
             w1_ref, b1_ref, w2_ref, b2_ref, out_ref):
        eye = (lax.broadcasted_iota(jnp.int32, (NC, NC), 0)
               == lax.broadcasted_iota(jnp.int32, (NC, NC), 1))
        a = p1a_ref[...] + eye.astype(jnp.float32)[None]
        deg = jnp.sum(a, axis=2, keepdims=True)
        dinv = lax.rsqrt(jnp.maximum(deg, 1e-12))
        an_rows = [dinv[g] * a[g] * dinv[g].reshape(1, NC) for g in range(B)]

        def dense_layer(h, w, bias):
            hw = _dot(h.reshape(B * NC, -1), w, _CT_STD) + bias
            hw = hw.reshape(B, NC, -1)
            rows = [_dot(an_rows[g], hw[g], _CT_STD)[None] for g in range(B)]
            return jnp.concatenate(rows, axis=0)

        x21 = dense_layer(p1x_ref[...], w21_ref[...], b21_ref[...])
        x21n = bn_full(x21, g21_ref[...], be21_ref[...])
        x22 = dense_layer(x21n, w22_ref[...], b22_ref[...])
        x22n = bn_full(x22, g22_ref[...], be22_ref[...])
        x23 = dense_layer(x22n, w23_ref[...], b23_ref[...])
        x23n = bn_full(x23, g23_ref[...], be23_ref[...])
        x2 = jnp.concatenate([x21n, x22n, x23n], axis=2)
        x2out = jnp.max(x2, axis=1)
        conv = jnp.concatenate([x1o_ref[...].reshape(B, 90), x2out], axis=1)
        h = _dot(conv, w1_ref[...], _CT_STD) + b1_ref[...]
        h = jnp.maximum(h, 0.0)
        out_ref[...] = _dot(h, w2_ref[...], _CT_STD) + b2_ref[...]

    args = [p1x, p1adj, x1out,
            p["W21"], p["b21"].reshape(1, 30),
            p["W22"], p["b22"].reshape(1, 30),
            p["W23"], p["b23"].reshape(1, 30),
            p["gn21"].reshape(1, 1, 30), p["ben21"].reshape(1, 1, 30),
            p["gn22"].reshape(1, 1, 30), p["ben22"].reshape(1, 1, 30),
            p["gn23"].reshape(1, 1, 30), p["ben23"].reshape(1, 1, 30),
            p["Wf1"], p["bf1"].reshape(1, 50),
            p["Wf2"], p["bf2"].reshape(1, 6)]
    return pl.pallas_call(
        body,
        out_shape=jax.ShapeDtypeStruct((B, 6), jnp.float32),
    )(*args)


def kernel(x, edge_index, edge_attr, params):
    p = params
    src = edge_index[0]
    dst = edge_index[1]
    adj, adj1 = _sc_build_adj(src, dst, edge_attr)

    x3t = x.reshape(B, NPG, 3).swapaxes(1, 2)

    x11, xs1, xq1, sg11, ss1, sq1 = _gcn_pair(
        x3t, x3t, adj, adj1, p["W11"], p["b11"], p["Wp11"], p["bp11"])
    x12, xs2, xq2, sg12, ss2, sq2 = _gcn_pair(
        x11, sg11, adj, adj1, p["W12"], p["b12"], p["Wp12"], p["bp12"],
        bn_x=(xs1, xq1, p["gn11"], p["ben11"]),
        bn_s=(ss1, sq1, p["gnp11"], p["benp11"]))
    x13, xs3, xq3, sg13, ss3, sq3 = _gcn_pair(
        x12, sg12, adj, adj1, p["W13"], p["b13"], p["Wp13"], p["bp13"],
        bn_x=(xs2, xq2, p["gn12"], p["ben12"]),
        bn_s=(ss2, sq2, p["gnp12"], p["benp12"]))

    parts = []
    for q in range(2, 9):
        ph = jnp.sin(jnp.arange(N * q, dtype=jnp.float32) * 0.37).reshape(N, q)
        parts.append(jax.nn.softmax(ph, axis=-1))
    bel0t = jnp.concatenate(parts, axis=-1).reshape(B, NPG, QTOT).swapaxes(1, 2)

    p1x, p1adj, x1out = _bp_pool(
        adj, bel0t, (x11, x12, x13),
        ((xs1, xq1), (xs2, xq2), (xs3, xq3)),
        (sg11, sg12, sg13),
        ((ss1, sq1), (ss2, sq2), (ss3, sq3)), p)

    out = _dense_head(p1x, p1adj, x1out, p)
    reg = jnp.zeros((1,), jnp.float32)
    return (out, reg)
